# baseline (speedup 1.0000x reference)
import jax
import jax.numpy as jnp
from jax import lax
from jax.experimental import pallas as pl
from jax.experimental.pallas import tpu as pltpu
from jax.experimental.pallas import tpu_sc as plsc

N_NODES = 100000
N_EDGES = 3200000
OUT_DIM = 16
CUTOFF = 5.0

NC = 2
NS = 16
NW = NC * NS
TILES = N_EDGES // 128
TPW = TILES // NW
EXTRA = TILES % NW
CHUNK = 1920
CT = CHUNK // 128
NMAIN = TPW // CT
TAILC = 128
GSUB = 128
NG = CHUNK // GSUB

WIDTH = CUTOFF / (OUT_DIM - 1)
NEG_S = -1.0 / (2.0 * WIDTH * WIDTH)
CENTERS = [k * WIDTH for k in range(OUT_DIM)]


def _rsqrt_nr(d2):
    d2c = jnp.maximum(d2, 1e-30)
    i = plsc.bitcast(d2c, jnp.int32)
    i = 0x5F3759DF - lax.shift_right_logical(i, 1)
    y = plsc.bitcast(i, jnp.float32)
    nh = d2c * -0.5
    for _ in range(3):
        y = y * (1.5 + nh * y * y)
    return y


def _sc_body(px_hbm, py_hbm, pz_hbm, src_hbm, dst_hbm, out_hbm,
             sidx0, sidx1, didx0, didx1,
             sx0, sx1, sy0, sy1, sz0, sz1, tx0, tx1, ty0, ty1, tz0, tz1,
             outv0, outv1, px_sh, py_sh, pz_sh,
             gsem0, gsem1, osem0, osem1):
    sidx = (sidx0, sidx1)
    didx = (didx0, didx1)
    sx = (sx0, sx1)
    sy = (sy0, sy1)
    sz = (sz0, sz1)
    tx = (tx0, tx1)
    ty = (ty0, ty1)
    tz = (tz0, tz1)
    outv = (outv0, outv1)
    gsem = (gsem0, gsem1)
    osem = (osem0, osem1)

    wid = lax.axis_index("s") * NC + lax.axis_index("c")
    sid = lax.axis_index("s")
    tb = wid * TPW + jnp.minimum(wid, EXTRA)
    ebase0 = tb * 128

    @pl.when(sid == 0)
    def _stage_pos():
        pltpu.sync_copy(px_hbm, px_sh)
        pltpu.sync_copy(py_hbm, py_sh)
        pltpu.sync_copy(pz_hbm, pz_sh)

    plsc.subcore_barrier()

    def stage_and_fire(b, base, n, ng):
        pltpu.sync_copy(src_hbm.at[pl.ds(base, n)], sidx[b].at[pl.ds(0, n)])
        pltpu.sync_copy(dst_hbm.at[pl.ds(base, n)], didx[b].at[pl.ds(0, n)])
        for j in range(ng):
            sl = pl.ds(j * GSUB, GSUB)
            pltpu.async_copy(px_sh.at[sidx[b].at[sl]], sx[b].at[sl], gsem[b])
            pltpu.async_copy(py_sh.at[sidx[b].at[sl]], sy[b].at[sl], gsem[b])
            pltpu.async_copy(pz_sh.at[sidx[b].at[sl]], sz[b].at[sl], gsem[b])
            pltpu.async_copy(px_sh.at[didx[b].at[sl]], tx[b].at[sl], gsem[b])
            pltpu.async_copy(py_sh.at[didx[b].at[sl]], ty[b].at[sl], gsem[b])
            pltpu.async_copy(pz_sh.at[didx[b].at[sl]], tz[b].at[sl], gsem[b])

    def drain_gathers(b, n):
        dsl = pl.ds(0, n)
        for buf in (sx, sy, sz, tx, ty, tz):
            pltpu.make_async_copy(px_hbm.at[pl.ds(0, n)],
                                  buf[b].at[dsl], gsem[b]).wait()

    def compute(b, ngrp):
        def grp_body(g, _):
            gsl = pl.ds(g * 16, 16)
            dx = sx[b][gsl] - tx[b][gsl]
            dy = sy[b][gsl] - ty[b][gsl]
            dz = sz[b][gsl] - tz[b][gsl]
            d2 = dx * dx + dy * dy + dz * dz
            norm = d2 * _rsqrt_nr(d2)
            for k in range(OUT_DIM):
                t = norm - CENTERS[k]
                outv[b][k, gsl] = jnp.exp(t * (t * NEG_S))
            return 0

        lax.fori_loop(0, ngrp, grp_body, 0)

    stage_and_fire(0, ebase0, CHUNK, NG)

    def main_body(i2, _):
        for b in (0, 1):
            i = i2 * 2 + b

            @pl.when(i < NMAIN - 1)
            def _fire_next():
                stage_and_fire(1 - b, ebase0 + (i + 1) * CHUNK, CHUNK, NG)

            drain_gathers(b, CHUNK)

            @pl.when(i >= 2)
            def _drain_store():
                pltpu.make_async_copy(out_hbm.at[:, pl.ds(0, CHUNK)],
                                      outv[b], osem[b]).wait()

            compute(b, CHUNK // 16)
            pltpu.async_copy(outv[b], out_hbm.at[:, pl.ds(ebase0 + i * CHUNK,
                                                          CHUNK)], osem[b])
        return 0

    lax.fori_loop(0, NMAIN // 2, main_body, 0)
    for b in (0, 1):
        pltpu.make_async_copy(out_hbm.at[:, pl.ds(0, CHUNK)],
                              outv[b], osem[b]).wait()

    ntail = (TPW - NMAIN * CT) + jnp.where(wid < EXTRA, 1, 0)
    tail0 = ebase0 + NMAIN * CHUNK

    def tail_body(t, _):
        base = tail0 + t * TAILC
        stage_and_fire(0, base, TAILC, 1)
        drain_gathers(0, TAILC)
        compute(0, TAILC // 16)
        pltpu.sync_copy(outv[0].at[:, pl.ds(0, TAILC)],
                        out_hbm.at[:, pl.ds(base, TAILC)])
        return 0

    lax.fori_loop(0, ntail, tail_body, 0)


@jax.jit
def _sc_rbf(px, py, pz, src, dst):
    mesh = plsc.VectorSubcoreMesh(core_axis_name="c", subcore_axis_name="s")
    idx_t = pltpu.VMEM((CHUNK,), jnp.int32)
    f_t = pltpu.VMEM((CHUNK,), jnp.float32)
    o_t = pltpu.VMEM((OUT_DIM, CHUNK), jnp.float32)
    return pl.kernel(
        _sc_body,
        out_type=jax.ShapeDtypeStruct((OUT_DIM, N_EDGES), jnp.float32),
        mesh=mesh,
        compiler_params=pltpu.CompilerParams(needs_layout_passes=False),
        scratch_types=[
            idx_t, idx_t, idx_t, idx_t,
            f_t, f_t, f_t, f_t, f_t, f_t, f_t, f_t, f_t, f_t, f_t, f_t,
            o_t, o_t,
            pltpu.VMEM_SHARED((N_NODES,), jnp.float32),
            pltpu.VMEM_SHARED((N_NODES,), jnp.float32),
            pltpu.VMEM_SHARED((N_NODES,), jnp.float32),
            pltpu.SemaphoreType.DMA, pltpu.SemaphoreType.DMA,
            pltpu.SemaphoreType.DMA, pltpu.SemaphoreType.DMA,
        ],
    )(px, py, pz, src, dst)


def kernel(pos, edge_index):
    out_t = _sc_rbf(pos[:, 0], pos[:, 1], pos[:, 2],
                    edge_index[0], edge_index[1])
    return out_t.T


# speedup vs baseline: 1.0105x; 1.0105x over previous
import jax
import jax.numpy as jnp
from jax import lax
from jax.experimental import pallas as pl
from jax.experimental.pallas import tpu as pltpu
from jax.experimental.pallas import tpu_sc as plsc

N_NODES = 100000
N_EDGES = 3200000
OUT_DIM = 16
CUTOFF = 5.0

NC = 2
NS = 16
NW = NC * NS
TILES = N_EDGES // 128
TPW = TILES // NW
EXTRA = TILES % NW
CHUNK = 1280
CT = CHUNK // 128
NMAIN = TPW // CT
TAILC = 128
GSUB = 128
NG = CHUNK // GSUB

WIDTH = CUTOFF / (OUT_DIM - 1)
NEG_S = -1.0 / (2.0 * WIDTH * WIDTH)
CENTERS = [k * WIDTH for k in range(OUT_DIM)]


def _rsqrt_nr(d2):
    d2c = jnp.maximum(d2, 1e-30)
    i = plsc.bitcast(d2c, jnp.int32)
    i = 0x5F3759DF - lax.shift_right_logical(i, 1)
    y = plsc.bitcast(i, jnp.float32)
    nh = d2c * -0.5
    for _ in range(3):
        y = y * (1.5 + nh * y * y)
    return y


def _sc_body(px_hbm, py_hbm, pz_hbm, src_hbm, dst_hbm, out_hbm,
             sidx0, sidx1, didx0, didx1,
             sx0, sx1, sy0, sy1, sz0, sz1, tx0, tx1, ty0, ty1, tz0, tz1,
             outv0, outv1, px_sh, py_sh, pz_sh,
             gsem0, gsem1, osem0, osem1):
    sidx = (sidx0, sidx1)
    didx = (didx0, didx1)
    sx = (sx0, sx1)
    sy = (sy0, sy1)
    sz = (sz0, sz1)
    tx = (tx0, tx1)
    ty = (ty0, ty1)
    tz = (tz0, tz1)
    outv = (outv0, outv1)
    gsem = (gsem0, gsem1)
    osem = (osem0, osem1)

    wid = lax.axis_index("s") * NC + lax.axis_index("c")
    sid = lax.axis_index("s")
    tb = wid * TPW + jnp.minimum(wid, EXTRA)
    ebase0 = tb * 128

    @pl.when(sid == 0)
    def _stage_pos():
        pltpu.sync_copy(px_hbm, px_sh)
        pltpu.sync_copy(py_hbm, py_sh)
        pltpu.sync_copy(pz_hbm, pz_sh)

    plsc.subcore_barrier()

    def stage_and_fire(b, base, n, ng):
        pltpu.sync_copy(src_hbm.at[pl.ds(base, n)], sidx[b].at[pl.ds(0, n)])
        pltpu.sync_copy(dst_hbm.at[pl.ds(base, n)], didx[b].at[pl.ds(0, n)])
        for j in range(ng):
            sl = pl.ds(j * GSUB, GSUB)
            pltpu.async_copy(px_sh.at[sidx[b].at[sl]], sx[b].at[sl], gsem[b])
            pltpu.async_copy(py_sh.at[sidx[b].at[sl]], sy[b].at[sl], gsem[b])
            pltpu.async_copy(pz_sh.at[sidx[b].at[sl]], sz[b].at[sl], gsem[b])
            pltpu.async_copy(px_sh.at[didx[b].at[sl]], tx[b].at[sl], gsem[b])
            pltpu.async_copy(py_sh.at[didx[b].at[sl]], ty[b].at[sl], gsem[b])
            pltpu.async_copy(pz_sh.at[didx[b].at[sl]], tz[b].at[sl], gsem[b])

    def drain_gathers(b, n):
        dsl = pl.ds(0, n)
        for buf in (sx, sy, sz, tx, ty, tz):
            pltpu.make_async_copy(px_hbm.at[pl.ds(0, n)],
                                  buf[b].at[dsl], gsem[b]).wait()

    def compute(b, ngrp):
        def grp_body(g, _):
            gsl = pl.ds(g * 16, 16)
            dx = sx[b][gsl] - tx[b][gsl]
            dy = sy[b][gsl] - ty[b][gsl]
            dz = sz[b][gsl] - tz[b][gsl]
            d2 = dx * dx + dy * dy + dz * dz
            norm = d2 * _rsqrt_nr(d2)
            for k in range(OUT_DIM):
                t = norm - CENTERS[k]
                outv[b][k, gsl] = jnp.exp(t * (t * NEG_S))
            return 0

        lax.fori_loop(0, ngrp, grp_body, 0)

    stage_and_fire(0, ebase0, CHUNK, NG)

    def main_body(i2, _):
        for b in (0, 1):
            i = i2 * 2 + b

            @pl.when(i < NMAIN - 1)
            def _fire_next():
                stage_and_fire(1 - b, ebase0 + (i + 1) * CHUNK, CHUNK, NG)

            drain_gathers(b, CHUNK)

            @pl.when(i >= 2)
            def _drain_store():
                pltpu.make_async_copy(out_hbm.at[:, pl.ds(0, CHUNK)],
                                      outv[b], osem[b]).wait()

            compute(b, CHUNK // 16)
            pltpu.async_copy(outv[b], out_hbm.at[:, pl.ds(ebase0 + i * CHUNK,
                                                          CHUNK)], osem[b])
        return 0

    lax.fori_loop(0, NMAIN // 2, main_body, 0)
    for b in (0, 1):
        pltpu.make_async_copy(out_hbm.at[:, pl.ds(0, CHUNK)],
                              outv[b], osem[b]).wait()

    ntail = (TPW - NMAIN * CT) + jnp.where(wid < EXTRA, 1, 0)
    tail0 = ebase0 + NMAIN * CHUNK

    def tail_body(t, _):
        base = tail0 + t * TAILC
        stage_and_fire(0, base, TAILC, 1)
        drain_gathers(0, TAILC)
        compute(0, TAILC // 16)
        pltpu.sync_copy(outv[0].at[:, pl.ds(0, TAILC)],
                        out_hbm.at[:, pl.ds(base, TAILC)])
        return 0

    lax.fori_loop(0, ntail, tail_body, 0)


@jax.jit
def _sc_rbf(px, py, pz, src, dst):
    mesh = plsc.VectorSubcoreMesh(core_axis_name="c", subcore_axis_name="s")
    idx_t = pltpu.VMEM((CHUNK,), jnp.int32)
    f_t = pltpu.VMEM((CHUNK,), jnp.float32)
    o_t = pltpu.VMEM((OUT_DIM, CHUNK), jnp.float32)
    return pl.kernel(
        _sc_body,
        out_type=jax.ShapeDtypeStruct((OUT_DIM, N_EDGES), jnp.float32),
        mesh=mesh,
        compiler_params=pltpu.CompilerParams(needs_layout_passes=False),
        scratch_types=[
            idx_t, idx_t, idx_t, idx_t,
            f_t, f_t, f_t, f_t, f_t, f_t, f_t, f_t, f_t, f_t, f_t, f_t,
            o_t, o_t,
            pltpu.VMEM_SHARED((N_NODES,), jnp.float32),
            pltpu.VMEM_SHARED((N_NODES,), jnp.float32),
            pltpu.VMEM_SHARED((N_NODES,), jnp.float32),
            pltpu.SemaphoreType.DMA, pltpu.SemaphoreType.DMA,
            pltpu.SemaphoreType.DMA, pltpu.SemaphoreType.DMA,
        ],
    )(px, py, pz, src, dst)


def kernel(pos, edge_index):
    out_t = _sc_rbf(pos[:, 0], pos[:, 1], pos[:, 2],
                    edge_index[0], edge_index[1])
    return out_t.T
